# baseline (speedup 1.0000x reference)
import functools

import jax
import jax.numpy as jnp
from jax import lax
from jax.experimental import pallas as pl
from jax.experimental.pallas import tpu as pltpu
from jax.experimental.pallas import tpu_sc as plsc

LANES = 16


def _make_plane_gather(batch, ncol, d_embed):
    mesh = plsc.VectorSubcoreMesh(core_axis_name="c", subcore_axis_name="s")
    bblk = batch // 128
    n_units = ncol * bblk
    n_workers = 32
    per_w = n_units // n_workers

    @functools.partial(
        pl.kernel,
        mesh=mesh,
        compiler_params=pltpu.CompilerParams(use_tc_tiling_on_sc=False),
        out_type=jax.ShapeDtypeStruct(
            (ncol, d_embed // 8, bblk, 8, 128), jnp.float32),
        scratch_types=[
            pltpu.VMEM((128,), jnp.int32),
            pltpu.VMEM((128,), jnp.int32),
            pltpu.VMEM((d_embed, 128), jnp.float32),
            pltpu.SemaphoreType.DMA,
        ],
    )
    def k(xt_hbm, wt_hbm, offb_hbm, out_hbm, off_v, idx_v, buf_v, sem_g):
        nc = lax.axis_index("c")
        ns = lax.axis_index("s")
        wid = ns * 2 + nc
        u0 = wid * per_w

        def body(i, _):
            u = u0 + i
            kcol = u // bblk
            bhi = u % bblk
            pltpu.sync_copy(offb_hbm.at[kcol], off_v)
            pltpu.sync_copy(xt_hbm.at[kcol, pl.ds(bhi * 128, 128)], idx_v)
            for t in range(8):
                idx_v[pl.ds(t * LANES, LANES)] = (
                    idx_v[pl.ds(t * LANES, LANES)]
                    + off_v[pl.ds(t * LANES, LANES)]
                )
            cps = [
                pltpu.async_copy(wt_hbm.at[e].at[idx_v], buf_v.at[e], sem_g)
                for e in range(d_embed)
            ]
            for cp in cps:
                cp.wait()
            for eh in range(d_embed // 8):
                pltpu.sync_copy(
                    buf_v.at[pl.ds(eh * 8, 8)], out_hbm.at[kcol, eh, bhi])
            return ()

        lax.fori_loop(0, per_w, body, (), unroll=False)

    return k


def kernel(x_cat, table, offsets):
    batch, ncol = x_cat.shape
    _, d_embed = table.shape
    offb = jnp.broadcast_to(
        offsets.astype(jnp.int32)[:, None], (ncol, 128))
    k = _make_plane_gather(batch, ncol, d_embed)
    out5 = k(x_cat.T, table.T, offb)
    return out5.transpose(2, 4, 0, 1, 3).reshape(batch, ncol, d_embed)


# speedup vs baseline: 1.8267x; 1.8267x over previous
import functools
import math

import jax
import jax.numpy as jnp
from jax import lax
from jax.experimental import pallas as pl
from jax.experimental.pallas import tpu as pltpu
from jax.experimental.pallas import tpu_sc as plsc

LANES = 16
N_WORKERS = 32


def _make_relayout(v_rows, d_embed):
    mesh = plsc.VectorSubcoreMesh(core_axis_name="c", subcore_axis_name="s")
    full = v_rows // 128
    tail = v_rows - full * 128
    base = full // N_WORKERS
    extra = full - base * N_WORKERS

    @functools.partial(
        pl.kernel,
        mesh=mesh,
        compiler_params=pltpu.CompilerParams(needs_layout_passes=False),
        out_type=jax.ShapeDtypeStruct((v_rows * d_embed,), jnp.float32),
        scratch_types=[
            pltpu.VMEM((8, 128), jnp.float32),
            pltpu.VMEM((128 * d_embed,), jnp.float32),
            pltpu.VMEM((LANES,), jnp.int32),
            pltpu.SemaphoreType.DMA,
        ],
    )
    def k1(wt_hbm, tail_hbm, out_hbm, src_v, dst_v, i32_v, sem):
        nc = lax.axis_index("c")
        ns = lax.axis_index("s")
        wid = ns * 2 + nc
        lo = wid * base + jnp.minimum(wid, extra)
        nblk = base + jnp.where(wid < extra, 1, 0)
        iota = jax.lax.iota(jnp.int32, LANES)
        i32_v[...] = iota * d_embed

        def do_block(c, rows):
            for a in range(d_embed // 8):
                pltpu.async_copy(
                    wt_hbm.at[pl.ds(a * 8, 8), pl.ds(c * 128, rows)],
                    src_v.at[:, pl.ds(0, rows)], sem).wait()
                for ep in range(8):
                    e = a * 8 + ep
                    for g in range(rows // LANES):
                        vec = src_v[ep, pl.ds(g * LANES, LANES)]
                        pos = i32_v[...] + (g * LANES * d_embed + e)
                        plsc.store_scatter(dst_v, [pos], vec)
            pltpu.sync_copy(
                dst_v.at[pl.ds(0, rows * d_embed)],
                out_hbm.at[pl.ds(c * 128 * d_embed, rows * d_embed)])

        def body(i, _):
            do_block(lo + i, 128)
            return ()

        lax.fori_loop(0, nblk, body, (), unroll=False)
        if tail:
            @pl.when(wid == N_WORKERS - 1)
            def _():
                n = tail * d_embed
                pltpu.sync_copy(tail_hbm, dst_v.at[pl.ds(0, n)])
                pltpu.sync_copy(
                    dst_v.at[pl.ds(0, n)],
                    out_hbm.at[pl.ds(full * 128 * d_embed, n)])

    return k1


def _make_gather(n_flat, d_embed, chunk, off_len):
    mesh = plsc.VectorSubcoreMesh(core_axis_name="c", subcore_axis_name="s")
    per_w = n_flat // N_WORKERS
    n_chunks = per_w // chunk

    @functools.partial(
        pl.kernel,
        mesh=mesh,
        compiler_params=pltpu.CompilerParams(use_tc_tiling_on_sc=False),
        out_type=jax.ShapeDtypeStruct((n_flat, d_embed), jnp.float32),
        scratch_types=[
            pltpu.VMEM((off_len,), jnp.int32),
            pltpu.VMEM((2, chunk), jnp.int32),
            pltpu.VMEM((2, chunk, d_embed), jnp.float32),
            pltpu.SemaphoreType.DMA,
            pltpu.SemaphoreType.DMA,
            pltpu.SemaphoreType.DMA,
            pltpu.SemaphoreType.DMA,
            pltpu.SemaphoreType.DMA,
        ],
    )
    def k2(idx_hbm, table_hbm, off_hbm, out_hbm, off_v, idx_v, rows_v,
           sem_g, sem_i0, sem_i1, sem_o0, sem_o1):
        nc = lax.axis_index("c")
        ns = lax.axis_index("s")
        wid = ns * 2 + nc
        pltpu.sync_copy(off_hbm, off_v)
        base0 = wid * per_w
        sem_i = (sem_i0, sem_i1)
        sem_o = (sem_o0, sem_o1)

        def load(c):
            b = c % 2
            return pltpu.async_copy(
                idx_hbm.at[pl.ds(base0 + c * chunk, chunk)], idx_v.at[b],
                sem_i[b])

        def add_offsets(c):
            b = c % 2
            for j in range(chunk // off_len):
                for t in range(off_len // LANES):
                    p = j * off_len + t * LANES
                    idx_v[b, pl.ds(p, LANES)] = (
                        idx_v[b, pl.ds(p, LANES)]
                        + off_v[pl.ds(t * LANES, LANES)]
                    )

        def gather(c):
            b = c % 2
            return pltpu.async_copy(
                table_hbm.at[idx_v.at[b]], rows_v.at[b], sem_g)

        def store(c):
            b = c % 2
            return pltpu.async_copy(
                rows_v.at[b], out_hbm.at[pl.ds(base0 + c * chunk, chunk)],
                sem_o[b])

        cp_l0 = load(0)
        cp_l1 = load(1)
        cp_l0.wait()
        add_offsets(0)
        cp_g = gather(0)
        cp_s = [None, None]
        cp_l = [None, cp_l1]
        for c in range(n_chunks):
            b = c % 2
            nb = (c + 1) % 2
            if c + 1 < n_chunks:
                cp_l[nb].wait()
                add_offsets(c + 1)
            cp_g.wait()
            cp_s[b] = store(c)
            if c + 2 < n_chunks:
                cp_l[b] = load(c + 2)
            if c + 1 < n_chunks:
                if cp_s[nb] is not None:
                    cp_s[nb].wait()
                cp_g = gather(c + 1)
        if cp_s[(n_chunks - 2) % 2] is not None:
            cp_s[(n_chunks - 2) % 2].wait()
        cp_s[(n_chunks - 1) % 2].wait()

    return k2


def kernel(x_cat, table, offsets):
    batch, ncol = x_cat.shape
    v_rows, d_embed = table.shape
    n_flat = batch * ncol
    off_len = ncol * LANES // math.gcd(ncol, LANES)
    idx_flat = x_cat.reshape(n_flat).astype(jnp.int32)
    off_tiled = jnp.tile(offsets.astype(jnp.int32), off_len // ncol)
    chunk = 1664
    assert n_flat % (N_WORKERS * chunk) == 0 and chunk % off_len == 0
    k1 = _make_relayout(v_rows, d_embed)
    tail_rows = (v_rows // 128) * 128
    tail64 = table[tail_rows:].reshape(-1)
    t_flat = k1(table.T, tail64)
    t_rm = t_flat.reshape(v_rows, d_embed)
    k2 = _make_gather(n_flat, d_embed, chunk, off_len)
    out = k2(idx_flat, t_rm, off_tiled)
    return out.reshape(batch, ncol, d_embed)


# speedup vs baseline: 3.8053x; 2.0832x over previous
import functools
import math

import jax
import jax.numpy as jnp
from jax import lax
from jax.experimental import pallas as pl
from jax.experimental.pallas import tpu as pltpu
from jax.experimental.pallas import tpu_sc as plsc

LANES = 16
N_WORKERS = 32


def _make_relayout(v_rows, d_embed):
    mesh = plsc.VectorSubcoreMesh(core_axis_name="c", subcore_axis_name="s")
    full = v_rows // 128
    tail = v_rows - full * 128
    per_w = (full + N_WORKERS - 1) // N_WORKERS
    while per_w % 3:
        per_w += 1
    n_iter = per_w // 3

    @functools.partial(
        pl.kernel,
        mesh=mesh,
        compiler_params=pltpu.CompilerParams(needs_layout_passes=False),
        out_type=jax.ShapeDtypeStruct((v_rows * d_embed // 128, 128),
                                      jnp.float32),
        scratch_types=[
            pltpu.VMEM((d_embed, 128), jnp.float32),
            pltpu.VMEM((d_embed, 128), jnp.float32),
            pltpu.VMEM((d_embed, 128), jnp.float32),
            pltpu.VMEM((32, 133), jnp.float32),
            pltpu.VMEM((32, 133), jnp.float32),
            pltpu.VMEM((32, 133), jnp.float32),
            pltpu.SemaphoreType.DMA,
            pltpu.SemaphoreType.DMA,
            pltpu.SemaphoreType.DMA,
            pltpu.SemaphoreType.DMA,
            pltpu.SemaphoreType.DMA,
            pltpu.SemaphoreType.DMA,
        ],
    )
    def k1(wt_hbm, tail_hbm, out_hbm, src0_v, src1_v, src2_v,
           dst0_v, dst1_v, dst2_v, si0, si1, si2, so0, so1, so2):
        nc = lax.axis_index("c")
        ns = lax.axis_index("s")
        wid = ns * 2 + nc
        lo = wid * per_w
        sem_i = (si0, si1, si2)
        sem_o = (so0, so1, so2)
        srcs = (src0_v, src1_v, src2_v)
        dsts = (dst0_v, dst1_v, dst2_v)

        def load(c, b):
            cc = jnp.minimum(c, full - 1)
            pltpu.async_copy(
                wt_hbm.at[:, pl.ds(cc * 128, 128)], srcs[b], sem_i[b])

        def wait_in(b):
            pltpu.make_async_copy(
                wt_hbm.at[:, pl.ds(0, 128)], srcs[b], sem_i[b]).wait()

        def wait_out(b):
            pltpu.make_async_copy(
                dsts[b].at[:, pl.ds(0, 128)],
                out_hbm.at[pl.ds(0, 32), :], sem_o[b]).wait()

        def transpose(b):
            base = jax.lax.iota(jnp.int32, LANES) * d_embed
            for e in range(d_embed):
                for g in range(128 // LANES):
                    pos = base + (g * LANES * d_embed + e)
                    row = jax.lax.shift_right_logical(pos, 7)
                    col = jax.lax.bitwise_and(pos, 127)
                    vec = srcs[b][e, pl.ds(g * LANES, LANES)]
                    plsc.store_scatter(dsts[b], [row, col], vec)

        def store(c, b):
            cc = jnp.minimum(c, full - 1)
            pltpu.async_copy(
                dsts[b].at[:, pl.ds(0, 128)],
                out_hbm.at[pl.ds(cc * 32, 32), :], sem_o[b])

        load(lo, 0)
        load(lo + 1, 1)
        load(lo + 2, 2)

        def body(i, _):
            for b in range(3):
                c = lo + 3 * i + b
                wait_in(b)

                @pl.when(i > 0)
                def _():
                    wait_out(b)

                transpose(b)

                @pl.when(3 * i + b + 3 < per_w)
                def _():
                    load(c + 3, b)

                store(c, b)
            return ()

        lax.fori_loop(0, n_iter, body, (), unroll=False)
        wait_out(0)
        wait_out(1)
        wait_out(2)

        if tail:
            @pl.when(wid == N_WORKERS - 1)
            def _():
                trows = tail * d_embed // 128
                pltpu.sync_copy(
                    tail_hbm, dst0_v.at[pl.ds(0, trows), pl.ds(0, 128)])
                pltpu.sync_copy(
                    dst0_v.at[pl.ds(0, trows), pl.ds(0, 128)],
                    out_hbm.at[pl.ds(full * d_embed // 4, trows), :])

    return k1


def _make_gather(n_flat, d_embed, chunk, off_len):
    mesh = plsc.VectorSubcoreMesh(core_axis_name="c", subcore_axis_name="s")
    per_w = n_flat // N_WORKERS
    n_chunks = per_w // chunk

    @functools.partial(
        pl.kernel,
        mesh=mesh,
        compiler_params=pltpu.CompilerParams(use_tc_tiling_on_sc=False),
        out_type=jax.ShapeDtypeStruct((n_flat, d_embed), jnp.float32),
        scratch_types=[
            pltpu.VMEM((off_len,), jnp.int32),
            pltpu.VMEM((2, chunk), jnp.int32),
            pltpu.VMEM((2, chunk, d_embed), jnp.float32),
            pltpu.SemaphoreType.DMA,
            pltpu.SemaphoreType.DMA,
            pltpu.SemaphoreType.DMA,
            pltpu.SemaphoreType.DMA,
            pltpu.SemaphoreType.DMA,
        ],
    )
    def k2(idx_hbm, table_hbm, off_hbm, out_hbm, off_v, idx_v, rows_v,
           sem_g, sem_i0, sem_i1, sem_o0, sem_o1):
        nc = lax.axis_index("c")
        ns = lax.axis_index("s")
        wid = ns * 2 + nc
        pltpu.sync_copy(off_hbm, off_v)
        base0 = wid * per_w
        sem_i = (sem_i0, sem_i1)
        sem_o = (sem_o0, sem_o1)

        def load(c):
            b = c % 2
            return pltpu.async_copy(
                idx_hbm.at[pl.ds(base0 + c * chunk, chunk)], idx_v.at[b],
                sem_i[b])

        def add_offsets(c):
            b = c % 2
            for j in range(chunk // off_len):
                for t in range(off_len // LANES):
                    p = j * off_len + t * LANES
                    idx_v[b, pl.ds(p, LANES)] = (
                        idx_v[b, pl.ds(p, LANES)]
                        + off_v[pl.ds(t * LANES, LANES)]
                    )

        def gather(c):
            b = c % 2
            return pltpu.async_copy(
                table_hbm.at[idx_v.at[b]], rows_v.at[b], sem_g)

        def store(c):
            b = c % 2
            return pltpu.async_copy(
                rows_v.at[b], out_hbm.at[pl.ds(base0 + c * chunk, chunk)],
                sem_o[b])

        cp_l0 = load(0)
        cp_l1 = load(1)
        cp_l0.wait()
        add_offsets(0)
        cp_g = gather(0)
        cp_s = [None, None]
        cp_l = [None, cp_l1]
        for c in range(n_chunks):
            b = c % 2
            nb = (c + 1) % 2
            if c + 1 < n_chunks:
                cp_l[nb].wait()
                add_offsets(c + 1)
            cp_g.wait()
            cp_s[b] = store(c)
            if c + 2 < n_chunks:
                cp_l[b] = load(c + 2)
            if c + 1 < n_chunks:
                if cp_s[nb] is not None:
                    cp_s[nb].wait()
                cp_g = gather(c + 1)
        if cp_s[(n_chunks - 2) % 2] is not None:
            cp_s[(n_chunks - 2) % 2].wait()
        cp_s[(n_chunks - 1) % 2].wait()

    return k2


def kernel(x_cat, table, offsets):
    batch, ncol = x_cat.shape
    v_rows, d_embed = table.shape
    n_flat = batch * ncol
    off_len = ncol * LANES // math.gcd(ncol, LANES)
    idx_flat = x_cat.reshape(n_flat).astype(jnp.int32)
    off_tiled = jnp.tile(offsets.astype(jnp.int32), off_len // ncol)
    chunk = 1664
    assert n_flat % (N_WORKERS * chunk) == 0 and chunk % off_len == 0
    k1 = _make_relayout(v_rows, d_embed)
    tail_rows = (v_rows // 128) * 128
    tail64 = table[tail_rows:].reshape(-1, 128)
    t128 = k1(table.T, tail64)
    t_rm = t128.reshape(v_rows, d_embed)
    k2 = _make_gather(n_flat, d_embed, chunk, off_len)
    out = k2(idx_flat, t_rm, off_tiled)
    return out.reshape(batch, ncol, d_embed)
